# baseline (speedup 1.0000x reference)
import functools

import jax
import jax.numpy as jnp
from jax import lax
from jax.experimental import pallas as pl
from jax.experimental.pallas import tpu as pltpu
from jax.experimental.pallas import tpu_sc as plsc

N = 10000
E = 320000
D = 128
EPS = 1e-5

NC = 2
NS = 16
NW = NC * NS
EPW = E // NW
K = 80
N_PAD = 10240
TROWS = N_PAD // NS

_MESH = plsc.VectorSubcoreMesh(
    core_axis_name="c", subcore_axis_name="s", num_cores=NC, num_subcores=NS
)


HNI = 8


@functools.partial(
    pl.kernel,
    out_type=jax.ShapeDtypeStruct((NC, N_PAD), jnp.float32),
    mesh=_MESH,
    scratch_types=[
        [pltpu.VMEM((K,), jnp.int32)] * HNI,
        pltpu.VMEM((K,), jnp.float32),
        pltpu.VMEM((TROWS,), jnp.float32),
        pltpu.VMEM_SHARED((N_PAD,), jnp.float32),
        pltpu.SemaphoreType.DMA,
        pltpu.SemaphoreType.DMA,
    ],
)
def _hist_sc(ei_hbm, out_hbm, idx_v, ones_v, zeros_v, hist_sh, sem_i, sem_s):
    cid = lax.axis_index("c")
    sid = lax.axis_index("s")
    wid = sid * NC + cid

    def fill_ones(i, _):
        ones_v[pl.ds(i * 16, 16)] = jnp.ones((16,), jnp.float32)
        return 0

    lax.fori_loop(0, K // 16, fill_ones, 0)

    def fill_zeros(i, _):
        zeros_v[pl.ds(i * 16, 16)] = jnp.zeros((16,), jnp.float32)
        return 0

    lax.fori_loop(0, TROWS // 16, fill_zeros, 0)
    pltpu.sync_copy(zeros_v, hist_sh.at[pl.ds(sid * TROWS, TROWS)])
    plsc.subcore_barrier()

    nch = EPW // K

    def drain_one(slot):
        pltpu.make_async_copy(ei_hbm.at[pl.ds(0, K)], idx_v[slot], sem_s).wait()

    def fire_idx(c, slot):
        pltpu.async_copy(
            ei_hbm.at[pl.ds(E + wid * EPW + c * K, K)], idx_v[slot], sem_i
        )

    fire_idx(0, 0)
    fire_idx(1, 1)

    def step(c, u):
        @pl.when(jnp.logical_and(c >= 4, c - 4 <= nch - 1))
        def _():
            drain_one((u - 4) % HNI)

        @pl.when(c + 2 <= nch - 1)
        def _():
            fire_idx(c + 2, (u + 2) % HNI)

        @pl.when(c <= nch - 1)
        def _():
            pltpu.make_async_copy(
                ei_hbm.at[pl.ds(0, K)], idx_v[u], sem_i
            ).wait()
            pltpu.async_copy(ones_v, hist_sh.at[idx_v[u]], sem_s, add=True)

    def octet(it, _):
        for u in range(HNI):
            step(it * HNI + u, u)
        return 0

    lax.fori_loop(0, (nch + 2 + HNI - 1) // HNI, octet, 0)
    drain_one((nch - 1) % HNI)
    plsc.subcore_barrier()
    pltpu.sync_copy(
        hist_sh.at[pl.ds(sid * TROWS, TROWS)],
        out_hbm.at[cid, pl.ds(sid * TROWS, TROWS)],
    )


NR = 4
NI = 8
NCH = EPW // K


@functools.partial(
    pl.kernel,
    out_type=jax.ShapeDtypeStruct((NC, N_PAD, D), jnp.float32),
    mesh=_MESH,
    scratch_types=[
        [pltpu.VMEM((K,), jnp.int32)] * NI,
        [pltpu.VMEM((K,), jnp.int32)] * NI,
        [pltpu.VMEM((K, D), jnp.float32)] * NR,
        pltpu.VMEM((16, D), jnp.float32),
        pltpu.VMEM_SHARED((N_PAD, D), jnp.float32),
        pltpu.SemaphoreType.DMA,
        [pltpu.SemaphoreType.DMA] * NR,
        pltpu.SemaphoreType.DMA,
    ],
)
def _segsum_sc(ei_hbm, g_hbm, out_hbm, si_v, di_v, rows_v, zrows_v,
               acc_sh, sem_i, sem_g, sem_s):
    cid = lax.axis_index("c")
    sid = lax.axis_index("s")
    wid = sid * NC + cid
    row0 = sid * TROWS

    @pl.when(cid == 0)
    def _():
        def initg(i, _):
            pltpu.sync_copy(
                g_hbm.at[pl.ds(row0 + i * 64, 64)],
                acc_sh.at[pl.ds(row0 + i * 64, 64)],
            )
            return 0

        lax.fori_loop(0, TROWS // 64, initg, 0)

    @pl.when(cid != 0)
    def _():
        def fillz(i, _):
            def fillz16(j, _):
                zrows_v[i, pl.ds(j * 16, 16)] = jnp.zeros((16,), jnp.float32)
                return 0

            lax.fori_loop(0, D // 16, fillz16, 0)
            return 0

        lax.fori_loop(0, 16, fillz, 0)

        def initz(i, _):
            pltpu.sync_copy(zrows_v, acc_sh.at[pl.ds(row0 + i * 16, 16)])
            return 0

        lax.fori_loop(0, TROWS // 16, initz, 0)

    plsc.subcore_barrier()

    def drain_scatter(r):
        pltpu.make_async_copy(g_hbm.at[pl.ds(0, K)], rows_v[r], sem_s).wait()

    def fire_idx(c, slot):
        base = wid * EPW + c * K
        pltpu.async_copy(ei_hbm.at[pl.ds(base, K)], si_v[slot], sem_i)
        pltpu.async_copy(ei_hbm.at[pl.ds(E + base, K)], di_v[slot], sem_i)

    def wait_idx(slot):
        pltpu.make_async_copy(ei_hbm.at[pl.ds(0, K)], si_v[slot], sem_i).wait()
        pltpu.make_async_copy(ei_hbm.at[pl.ds(0, K)], di_v[slot], sem_i).wait()

    fire_idx(0, 0)
    fire_idx(1, 1)

    UNROLL = NI

    def step(c, u):
        r = u % NR
        @pl.when(jnp.logical_and(c >= NR, c - NR <= NCH - 1))
        def _():
            drain_scatter(r)

        @pl.when(c + 2 <= NCH - 1)
        def _():
            fire_idx(c + 2, (u + 2) % UNROLL)

        @pl.when(c <= NCH - 1)
        def _():
            wait_idx(u)
            pltpu.async_copy(g_hbm.at[si_v[u]], rows_v[r], sem_g[r])

        @pl.when(jnp.logical_and(c >= 2, c - 2 <= NCH - 1))
        def _():
            rp = (u - 2) % NR
            pltpu.make_async_copy(
                g_hbm.at[pl.ds(0, K)], rows_v[rp], sem_g[rp]
            ).wait()
            pltpu.async_copy(
                rows_v[rp], acc_sh.at[di_v[(u - 2) % UNROLL]], sem_s, add=True
            )

    def octet(it, _):
        for u in range(UNROLL):
            step(it * UNROLL + u, u)
        return 0

    lax.fori_loop(0, (NCH + 2 + UNROLL - 1) // UNROLL, octet, 0)
    drain_scatter((NCH - 1) % NR)
    plsc.subcore_barrier()
    pltpu.sync_copy(
        acc_sh.at[pl.ds(row0, TROWS)], out_hbm.at[cid, pl.ds(row0, TROWS)]
    )


def _g_tc(x_ref, w_ref, hist_ref, g_ref):
    dis = lax.rsqrt(hist_ref[0] + hist_ref[1] + 1.0)
    h = jnp.dot(x_ref[...], w_ref[...], preferred_element_type=jnp.float32)
    g_ref[0:N, :] = dis[0:N] * h
    g_ref[N:N_PAD, :] = jnp.zeros((N_PAD - N, D), jnp.float32)


def _final_tc(acc_ref, hist_ref, b_ref, gamma_ref, beta_ref, out_ref):
    dis = lax.rsqrt(hist_ref[0] + hist_ref[1] + 1.0)
    t = dis[0:N] * (acc_ref[0, 0:N, :] + acc_ref[1, 0:N, :]) + b_ref[...]
    mean = jnp.mean(t, axis=0, keepdims=True)
    var = jnp.mean((t - mean) ** 2, axis=0, keepdims=True)
    o = (t - mean) * lax.rsqrt(var + EPS) * gamma_ref[...] + beta_ref[...]
    out_ref[...] = jnp.maximum(o, 0.0)


def kernel(x, edge_index, W, b, gamma, beta):
    ei_flat = edge_index.reshape(2 * E)
    hist3 = _hist_sc(ei_flat).reshape(NC, N_PAD, 1)
    g = pl.pallas_call(
        _g_tc, out_shape=jax.ShapeDtypeStruct((N_PAD, D), jnp.float32)
    )(x, W, hist3)
    acc = _segsum_sc(ei_flat, g)
    out = pl.pallas_call(
        _final_tc, out_shape=jax.ShapeDtypeStruct((N, D), jnp.float32)
    )(
        acc,
        hist3,
        b.reshape(1, D),
        gamma.reshape(1, D),
        beta.reshape(1, D),
    )
    return out


# speedup vs baseline: 1.0832x; 1.0832x over previous
import functools

import jax
import jax.numpy as jnp
from jax import lax
from jax.experimental import pallas as pl
from jax.experimental.pallas import tpu as pltpu
from jax.experimental.pallas import tpu_sc as plsc

N = 10000
E = 320000
D = 128
EPS = 1e-5

NC = 2
NS = 16
NW = NC * NS
EPW = E // NW
K = 80
N_PAD = 10240
TROWS = N_PAD // NS

_MESH = plsc.VectorSubcoreMesh(
    core_axis_name="c", subcore_axis_name="s", num_cores=NC, num_subcores=NS
)


HNI = 8
HK = 128
HF = EPW // HK
HT = EPW - HF * HK


@functools.partial(
    pl.kernel,
    out_type=jax.ShapeDtypeStruct((NC, N_PAD), jnp.float32),
    mesh=_MESH,
    scratch_types=[
        [pltpu.VMEM((HK,), jnp.int32)] * HNI,
        pltpu.VMEM((HT,), jnp.int32),
        pltpu.VMEM((HK,), jnp.float32),
        pltpu.VMEM((TROWS,), jnp.float32),
        pltpu.VMEM_SHARED((N_PAD,), jnp.float32),
        pltpu.SemaphoreType.DMA,
        pltpu.SemaphoreType.DMA,
    ],
)
def _hist_sc(ei_hbm, out_hbm, idx_v, tidx_v, ones_v, zeros_v, hist_sh,
             sem_i, sem_s):
    cid = lax.axis_index("c")
    sid = lax.axis_index("s")
    wid = sid * NC + cid

    def fill_ones(i, _):
        ones_v[pl.ds(i * 16, 16)] = jnp.ones((16,), jnp.float32)
        return 0

    lax.fori_loop(0, HK // 16, fill_ones, 0)

    def fill_zeros(i, _):
        zeros_v[pl.ds(i * 16, 16)] = jnp.zeros((16,), jnp.float32)
        return 0

    lax.fori_loop(0, TROWS // 16, fill_zeros, 0)
    pltpu.sync_copy(zeros_v, hist_sh.at[pl.ds(sid * TROWS, TROWS)])
    plsc.subcore_barrier()

    def drain_one(slot):
        pltpu.make_async_copy(ei_hbm.at[pl.ds(0, HK)], idx_v[slot], sem_s).wait()

    def fire_idx(c, slot):
        pltpu.async_copy(
            ei_hbm.at[pl.ds(E + wid * EPW + c * HK, HK)], idx_v[slot], sem_i
        )

    fire_idx(0, 0)
    fire_idx(1, 1)

    def step(c, u):
        @pl.when(jnp.logical_and(c >= 4, c - 4 <= HF - 1))
        def _():
            drain_one((u - 4) % HNI)

        @pl.when(c + 2 <= HF - 1)
        def _():
            fire_idx(c + 2, (u + 2) % HNI)

        @pl.when(c <= HF - 1)
        def _():
            pltpu.make_async_copy(
                ei_hbm.at[pl.ds(0, HK)], idx_v[u], sem_i
            ).wait()
            pltpu.async_copy(ones_v, hist_sh.at[idx_v[u]], sem_s, add=True)

    def octet(it, _):
        for u in range(HNI):
            step(it * HNI + u, u)
        return 0

    lax.fori_loop(0, (HF + 2 + HNI - 1) // HNI, octet, 0)
    for t in range(HF - 2, HF):
        drain_one(t % HNI)
    pltpu.sync_copy(
        ei_hbm.at[pl.ds(E + wid * EPW + HF * HK, HT)], tidx_v
    )
    pltpu.sync_copy(ones_v.at[pl.ds(0, HT)], hist_sh.at[tidx_v], add=True)
    plsc.subcore_barrier()
    pltpu.sync_copy(
        hist_sh.at[pl.ds(sid * TROWS, TROWS)],
        out_hbm.at[cid, pl.ds(sid * TROWS, TROWS)],
    )


NR = 4
NI = 8
NCH = EPW // K


@functools.partial(
    pl.kernel,
    out_type=jax.ShapeDtypeStruct((NC, N_PAD, D), jnp.float32),
    mesh=_MESH,
    scratch_types=[
        [pltpu.VMEM((K,), jnp.int32)] * NI,
        [pltpu.VMEM((K,), jnp.int32)] * NI,
        [pltpu.VMEM((K, D), jnp.float32)] * NR,
        pltpu.VMEM((16, D), jnp.float32),
        pltpu.VMEM_SHARED((N_PAD, D), jnp.float32),
        pltpu.SemaphoreType.DMA,
        [pltpu.SemaphoreType.DMA] * NR,
        pltpu.SemaphoreType.DMA,
    ],
)
def _segsum_sc(ei_hbm, g_hbm, out_hbm, si_v, di_v, rows_v, zrows_v,
               acc_sh, sem_i, sem_g, sem_s):
    cid = lax.axis_index("c")
    sid = lax.axis_index("s")
    wid = sid * NC + cid
    row0 = sid * TROWS

    @pl.when(cid == 0)
    def _():
        def initg(i, _):
            pltpu.sync_copy(
                g_hbm.at[pl.ds(row0 + i * 64, 64)],
                acc_sh.at[pl.ds(row0 + i * 64, 64)],
            )
            return 0

        lax.fori_loop(0, TROWS // 64, initg, 0)

    @pl.when(cid != 0)
    def _():
        def fillz(i, _):
            def fillz16(j, _):
                zrows_v[i, pl.ds(j * 16, 16)] = jnp.zeros((16,), jnp.float32)
                return 0

            lax.fori_loop(0, D // 16, fillz16, 0)
            return 0

        lax.fori_loop(0, 16, fillz, 0)

        def initz(i, _):
            pltpu.sync_copy(zrows_v, acc_sh.at[pl.ds(row0 + i * 16, 16)])
            return 0

        lax.fori_loop(0, TROWS // 16, initz, 0)

    plsc.subcore_barrier()

    def drain_scatter(r):
        pltpu.make_async_copy(g_hbm.at[pl.ds(0, K)], rows_v[r], sem_s).wait()

    def fire_idx(c, slot):
        base = wid * EPW + c * K
        pltpu.async_copy(ei_hbm.at[pl.ds(base, K)], si_v[slot], sem_i)
        pltpu.async_copy(ei_hbm.at[pl.ds(E + base, K)], di_v[slot], sem_i)

    def wait_idx(slot):
        pltpu.make_async_copy(ei_hbm.at[pl.ds(0, K)], si_v[slot], sem_i).wait()
        pltpu.make_async_copy(ei_hbm.at[pl.ds(0, K)], di_v[slot], sem_i).wait()

    fire_idx(0, 0)
    fire_idx(1, 1)

    UNROLL = NI

    def step(c, u):
        r = u % NR
        @pl.when(jnp.logical_and(c >= NR, c - NR <= NCH - 1))
        def _():
            drain_scatter(r)

        @pl.when(c + 2 <= NCH - 1)
        def _():
            fire_idx(c + 2, (u + 2) % UNROLL)

        @pl.when(c <= NCH - 1)
        def _():
            wait_idx(u)
            pltpu.async_copy(g_hbm.at[si_v[u]], rows_v[r], sem_g[r])

        @pl.when(jnp.logical_and(c >= 2, c - 2 <= NCH - 1))
        def _():
            rp = (u - 2) % NR
            pltpu.make_async_copy(
                g_hbm.at[pl.ds(0, K)], rows_v[rp], sem_g[rp]
            ).wait()
            pltpu.async_copy(
                rows_v[rp], acc_sh.at[di_v[(u - 2) % UNROLL]], sem_s, add=True
            )

    def octet(it, _):
        for u in range(UNROLL):
            step(it * UNROLL + u, u)
        return 0

    lax.fori_loop(0, (NCH + 2 + UNROLL - 1) // UNROLL, octet, 0)
    drain_scatter((NCH - 1) % NR)
    plsc.subcore_barrier()
    pltpu.sync_copy(
        acc_sh.at[pl.ds(row0, TROWS)], out_hbm.at[cid, pl.ds(row0, TROWS)]
    )


def _dis_tc(hist_ref, dis_ref):
    deg = hist_ref[0:1, :] + hist_ref[1:2, :] + 1.0
    dis_ref[...] = lax.rsqrt(deg)


def _g_tc(x_ref, w_ref, dis_ref, g_ref):
    h = jnp.dot(x_ref[...], w_ref[...], preferred_element_type=jnp.float32)
    g_ref[0:N, :] = dis_ref[0:N] * h
    g_ref[N:N_PAD, :] = jnp.zeros((N_PAD - N, D), jnp.float32)


def _final_tc(acc_ref, dis_ref, b_ref, gamma_ref, beta_ref, out_ref):
    t = dis_ref[0:N] * (acc_ref[0, 0:N, :] + acc_ref[1, 0:N, :]) + b_ref[...]
    mean = jnp.mean(t, axis=0, keepdims=True)
    var = jnp.mean((t - mean) ** 2, axis=0, keepdims=True)
    o = (t - mean) * lax.rsqrt(var + EPS) * gamma_ref[...] + beta_ref[...]
    out_ref[...] = jnp.maximum(o, 0.0)


def kernel(x, edge_index, W, b, gamma, beta):
    ei_flat = edge_index.reshape(2 * E)
    hist = _hist_sc(ei_flat)
    dis_row = pl.pallas_call(
        _dis_tc, out_shape=jax.ShapeDtypeStruct((1, N_PAD), jnp.float32)
    )(hist)
    dis_col = dis_row.reshape(N_PAD, 1)
    g = pl.pallas_call(
        _g_tc, out_shape=jax.ShapeDtypeStruct((N_PAD, D), jnp.float32)
    )(x, W, dis_col)
    acc = _segsum_sc(ei_flat, g)
    out = pl.pallas_call(
        _final_tc, out_shape=jax.ShapeDtypeStruct((N, D), jnp.float32)
    )(
        acc,
        dis_col,
        b.reshape(1, D),
        gamma.reshape(1, D),
        beta.reshape(1, D),
    )
    return out


# speedup vs baseline: 1.1220x; 1.0358x over previous
import functools

import jax
import jax.numpy as jnp
from jax import lax
from jax.experimental import pallas as pl
from jax.experimental.pallas import tpu as pltpu
from jax.experimental.pallas import tpu_sc as plsc

N = 10000
E = 320000
D = 128
EPS = 1e-5

NC = 2
NS = 16
NW = NC * NS
EPW = E // NW
K = 80
N_PAD = 10240
TROWS = N_PAD // NS

_MESH = plsc.VectorSubcoreMesh(
    core_axis_name="c", subcore_axis_name="s", num_cores=NC, num_subcores=NS
)


HNI = 8
HK = 128
HF = EPW // HK
HT = EPW - HF * HK


@functools.partial(
    pl.kernel,
    out_type=jax.ShapeDtypeStruct((NC, N_PAD), jnp.float32),
    mesh=_MESH,
    scratch_types=[
        [pltpu.VMEM((HK,), jnp.int32)] * HNI,
        pltpu.VMEM((HT,), jnp.int32),
        pltpu.VMEM((HK,), jnp.float32),
        pltpu.VMEM((TROWS,), jnp.float32),
        pltpu.VMEM_SHARED((N_PAD,), jnp.float32),
        pltpu.SemaphoreType.DMA,
        pltpu.SemaphoreType.DMA,
    ],
)
def _hist_sc(ei_hbm, out_hbm, idx_v, tidx_v, ones_v, zeros_v, hist_sh,
             sem_i, sem_s):
    cid = lax.axis_index("c")
    sid = lax.axis_index("s")
    wid = sid * NC + cid

    def fill_ones(i, _):
        ones_v[pl.ds(i * 16, 16)] = jnp.ones((16,), jnp.float32)
        return 0

    lax.fori_loop(0, HK // 16, fill_ones, 0)

    def fill_zeros(i, _):
        zeros_v[pl.ds(i * 16, 16)] = jnp.zeros((16,), jnp.float32)
        return 0

    lax.fori_loop(0, TROWS // 16, fill_zeros, 0)
    pltpu.sync_copy(zeros_v, hist_sh.at[pl.ds(sid * TROWS, TROWS)])
    plsc.subcore_barrier()

    def drain_one(slot):
        pltpu.make_async_copy(ei_hbm.at[pl.ds(0, HK)], idx_v[slot], sem_s).wait()

    def fire_idx(c, slot):
        pltpu.async_copy(
            ei_hbm.at[pl.ds(E + wid * EPW + c * HK, HK)], idx_v[slot], sem_i
        )

    fire_idx(0, 0)
    fire_idx(1, 1)

    def step(c, u):
        @pl.when(jnp.logical_and(c >= 4, c - 4 <= HF - 1))
        def _():
            drain_one((u - 4) % HNI)

        @pl.when(c + 2 <= HF - 1)
        def _():
            fire_idx(c + 2, (u + 2) % HNI)

        @pl.when(c <= HF - 1)
        def _():
            pltpu.make_async_copy(
                ei_hbm.at[pl.ds(0, HK)], idx_v[u], sem_i
            ).wait()
            pltpu.async_copy(ones_v, hist_sh.at[idx_v[u]], sem_s, add=True)

    def octet(it, _):
        for u in range(HNI):
            step(it * HNI + u, u)
        return 0

    lax.fori_loop(0, (HF + 2 + HNI - 1) // HNI, octet, 0)
    for t in range(HF - 2, HF):
        drain_one(t % HNI)
    pltpu.sync_copy(
        ei_hbm.at[pl.ds(E + wid * EPW + HF * HK, HT)], tidx_v
    )
    pltpu.sync_copy(ones_v.at[pl.ds(0, HT)], hist_sh.at[tidx_v], add=True)
    plsc.subcore_barrier()
    pltpu.sync_copy(
        hist_sh.at[pl.ds(sid * TROWS, TROWS)],
        out_hbm.at[cid, pl.ds(sid * TROWS, TROWS)],
    )


NR = 4
NI = 8
NCH = EPW // K


@functools.partial(
    pl.kernel,
    out_type=jax.ShapeDtypeStruct((NC, N_PAD, D), jnp.float32),
    mesh=_MESH,
    scratch_types=[
        [pltpu.VMEM((K,), jnp.int32)] * NI,
        [pltpu.VMEM((K,), jnp.int32)] * NI,
        [pltpu.VMEM((K, D), jnp.float32)] * NR,
        pltpu.VMEM((32, D), jnp.float32),
        pltpu.VMEM_SHARED((N_PAD, D), jnp.float32),
        pltpu.SemaphoreType.DMA,
        [pltpu.SemaphoreType.DMA] * NR,
        pltpu.SemaphoreType.DMA,
    ],
)
def _segsum_sc(ei_hbm, g_hbm, out_hbm, si_v, di_v, rows_v, zrows_v,
               acc_sh, sem_i, sem_g, sem_s):
    cid = lax.axis_index("c")
    sid = lax.axis_index("s")
    wid = sid * NC + cid
    row0 = sid * TROWS

    def fillz(i, _):
        def fillz16(j, _):
            zrows_v[i, pl.ds(j * 16, 16)] = jnp.zeros((16,), jnp.float32)
            return 0

        lax.fori_loop(0, D // 16, fillz16, 0)
        return 0

    lax.fori_loop(0, 32, fillz, 0)
    zds = [
        pltpu.async_copy(zrows_v, acc_sh.at[pl.ds(row0 + i * 32, 32)], sem_s)
        for i in range(TROWS // 32)
    ]
    for zd in zds:
        zd.wait()

    plsc.subcore_barrier()

    def drain_scatter(r):
        pltpu.make_async_copy(g_hbm.at[pl.ds(0, K)], rows_v[r], sem_s).wait()

    def fire_idx(c, slot):
        base = wid * EPW + c * K
        pltpu.async_copy(ei_hbm.at[pl.ds(base, K)], si_v[slot], sem_i)
        pltpu.async_copy(ei_hbm.at[pl.ds(E + base, K)], di_v[slot], sem_i)

    def wait_idx(slot):
        pltpu.make_async_copy(ei_hbm.at[pl.ds(0, K)], si_v[slot], sem_i).wait()
        pltpu.make_async_copy(ei_hbm.at[pl.ds(0, K)], di_v[slot], sem_i).wait()

    fire_idx(0, 0)
    fire_idx(1, 1)

    UNROLL = NI

    def step(c, u):
        r = u % NR
        @pl.when(jnp.logical_and(c >= NR, c - NR <= NCH - 1))
        def _():
            drain_scatter(r)

        @pl.when(c + 2 <= NCH - 1)
        def _():
            fire_idx(c + 2, (u + 2) % UNROLL)

        @pl.when(c <= NCH - 1)
        def _():
            wait_idx(u)
            pltpu.async_copy(g_hbm.at[si_v[u]], rows_v[r], sem_g[r])

        @pl.when(jnp.logical_and(c >= 2, c - 2 <= NCH - 1))
        def _():
            rp = (u - 2) % NR
            pltpu.make_async_copy(
                g_hbm.at[pl.ds(0, K)], rows_v[rp], sem_g[rp]
            ).wait()
            pltpu.async_copy(
                rows_v[rp], acc_sh.at[di_v[(u - 2) % UNROLL]], sem_s, add=True
            )

    def octet(it, _):
        for u in range(UNROLL):
            step(it * UNROLL + u, u)
        return 0

    lax.fori_loop(0, (NCH + 2 + UNROLL - 1) // UNROLL, octet, 0)
    drain_scatter((NCH - 1) % NR)
    plsc.subcore_barrier()
    pltpu.sync_copy(
        acc_sh.at[pl.ds(row0, TROWS)], out_hbm.at[cid, pl.ds(row0, TROWS)]
    )


def _dis_tc(hist_ref, dis_ref):
    deg = hist_ref[0:1, :] + hist_ref[1:2, :] + 1.0
    dis_ref[...] = lax.rsqrt(deg)


def _g_tc(x_ref, w_ref, dis_ref, g_ref):
    h = jnp.dot(x_ref[...], w_ref[...], preferred_element_type=jnp.float32)
    g_ref[0:N, :] = dis_ref[0:N] * h
    g_ref[N:N_PAD, :] = jnp.zeros((N_PAD - N, D), jnp.float32)


def _final_tc(acc_ref, g_ref, dis_ref, b_ref, gamma_ref, beta_ref, out_ref):
    a = acc_ref[0, 0:N, :] + acc_ref[1, 0:N, :] + g_ref[0:N, :]
    t = dis_ref[0:N] * a + b_ref[...]
    mean = jnp.mean(t, axis=0, keepdims=True)
    var = jnp.mean((t - mean) ** 2, axis=0, keepdims=True)
    o = (t - mean) * lax.rsqrt(var + EPS) * gamma_ref[...] + beta_ref[...]
    out_ref[...] = jnp.maximum(o, 0.0)


def kernel(x, edge_index, W, b, gamma, beta):
    ei_flat = edge_index.reshape(2 * E)
    hist = _hist_sc(ei_flat)
    dis_row = pl.pallas_call(
        _dis_tc, out_shape=jax.ShapeDtypeStruct((1, N_PAD), jnp.float32)
    )(hist)
    dis_col = dis_row.reshape(N_PAD, 1)
    g = pl.pallas_call(
        _g_tc, out_shape=jax.ShapeDtypeStruct((N_PAD, D), jnp.float32)
    )(x, W, dis_col)
    acc = _segsum_sc(ei_flat, g)
    out = pl.pallas_call(
        _final_tc, out_shape=jax.ShapeDtypeStruct((N, D), jnp.float32)
    )(
        acc,
        g,
        dis_col,
        b.reshape(1, D),
        gamma.reshape(1, D),
        beta.reshape(1, D),
    )
    return out
